# Initial kernel scaffold; baseline (speedup 1.0000x reference)
import jax
import jax.numpy as jnp
from jax.experimental import pallas as pl


def kernel(atom_state, bond_state, connectivity, bond_transform):
    raise NotImplementedError("write your pallas kernel here")


# speedup vs baseline: 5.3943x; 5.3943x over previous
import functools

import jax
import jax.numpy as jnp
from jax.experimental import pallas as pl


B, N, E, ATOM_DIM, BOND_DIM = 64, 128, 256, 64, 16


def _bmm_kernel(atom_ref, bond_ref, src_ref, tgt_ref, w_ref, out_ref):
    atom = atom_ref[0]
    bond = bond_ref[0]
    src = src_ref[0, 0]
    tgt = tgt_ref[0, 0]
    w = w_ref[...]

    iota_n = jax.lax.broadcasted_iota(jnp.int32, (E, N), 1)
    oh_src = (iota_n == src[:, None]).astype(jnp.float32)
    src_atoms = jax.lax.dot(oh_src, atom,
                            preferred_element_type=jnp.float32)

    v = jax.lax.dot(src_atoms, w, preferred_element_type=jnp.float32)

    msg = v[:, 0:ATOM_DIM] * bond[:, 0:1]
    for k in range(1, BOND_DIM):
        msg += v[:, k * ATOM_DIM:(k + 1) * ATOM_DIM] * bond[:, k:k + 1]

    iota_t = jax.lax.broadcasted_iota(jnp.int32, (N, E), 0)
    oh_tgt = (iota_t == tgt[None, :]).astype(jnp.float32)
    out_ref[0] = jax.lax.dot(oh_tgt, msg,
                             preferred_element_type=jnp.float32)


@jax.jit
def kernel(atom_state, bond_state, connectivity, bond_transform):
    w = bond_transform.reshape(BOND_DIM, ATOM_DIM, ATOM_DIM)
    w = w.transpose(2, 0, 1).reshape(ATOM_DIM, BOND_DIM * ATOM_DIM)
    src_idx = connectivity[:, :, 0].reshape(B, 1, E)
    tgt_idx = connectivity[:, :, 1].reshape(B, 1, E)

    return pl.pallas_call(
        _bmm_kernel,
        grid=(B,),
        in_specs=[
            pl.BlockSpec((1, N, ATOM_DIM), lambda b: (b, 0, 0)),
            pl.BlockSpec((1, E, BOND_DIM), lambda b: (b, 0, 0)),
            pl.BlockSpec((1, 1, E), lambda b: (b, 0, 0)),
            pl.BlockSpec((1, 1, E), lambda b: (b, 0, 0)),
            pl.BlockSpec((ATOM_DIM, BOND_DIM * ATOM_DIM), lambda b: (0, 0)),
        ],
        out_specs=pl.BlockSpec((1, N, ATOM_DIM), lambda b: (b, 0, 0)),
        out_shape=jax.ShapeDtypeStruct((B, N, ATOM_DIM), jnp.float32),
    )(atom_state, bond_state, src_idx, tgt_idx, w)


# speedup vs baseline: 9.9997x; 1.8538x over previous
import jax
import jax.numpy as jnp
from jax.experimental import pallas as pl


B, N, E, ATOM_DIM, BOND_DIM = 64, 128, 256, 64, 16
C = 8


def _bmm_kernel(atom_ref, bond_ref, src_ref, tgt_ref, w_ref, r_ref, r2_ref,
                out_ref):
    w = w_ref[...]

    iota_n = jax.lax.broadcasted_iota(jnp.int32, (E, N), 1)
    gathered = []
    for c in range(C):
        oh_src = (iota_n == src_ref[c, 0][:, None]).astype(jnp.float32)
        gathered.append(jax.lax.dot(oh_src, atom_ref[c],
                                    preferred_element_type=jnp.float32))
    src_atoms = jnp.concatenate(gathered, axis=0)

    v = jax.lax.dot(src_atoms, w, preferred_element_type=jnp.float32)

    bond = bond_ref[...].reshape(C * E, BOND_DIM)
    bond_exp = jax.lax.dot(bond, r_ref[...],
                           preferred_element_type=jnp.float32)
    msg = jax.lax.dot(v * bond_exp, r2_ref[...],
                      preferred_element_type=jnp.float32)

    iota_t = jax.lax.broadcasted_iota(jnp.int32, (N, E), 0)
    for c in range(C):
        oh_tgt = (iota_t == tgt_ref[c, 0][None, :]).astype(jnp.float32)
        out_ref[c] = jax.lax.dot(oh_tgt, msg[c * E:(c + 1) * E],
                                 preferred_element_type=jnp.float32)


@jax.jit
def kernel(atom_state, bond_state, connectivity, bond_transform):
    w = bond_transform.reshape(BOND_DIM, ATOM_DIM, ATOM_DIM)
    w = w.transpose(2, 0, 1).reshape(ATOM_DIM, BOND_DIM * ATOM_DIM)
    src_idx = connectivity[:, :, 0].reshape(B, 1, E)
    tgt_idx = connectivity[:, :, 1].reshape(B, 1, E)
    r = jnp.repeat(jnp.eye(BOND_DIM, dtype=jnp.float32), ATOM_DIM, axis=1)
    r2 = jnp.tile(jnp.eye(ATOM_DIM, dtype=jnp.float32), (BOND_DIM, 1))

    return pl.pallas_call(
        _bmm_kernel,
        grid=(B // C,),
        in_specs=[
            pl.BlockSpec((C, N, ATOM_DIM), lambda b: (b, 0, 0)),
            pl.BlockSpec((C, E, BOND_DIM), lambda b: (b, 0, 0)),
            pl.BlockSpec((C, 1, E), lambda b: (b, 0, 0)),
            pl.BlockSpec((C, 1, E), lambda b: (b, 0, 0)),
            pl.BlockSpec((ATOM_DIM, BOND_DIM * ATOM_DIM), lambda b: (0, 0)),
            pl.BlockSpec((BOND_DIM, BOND_DIM * ATOM_DIM), lambda b: (0, 0)),
            pl.BlockSpec((BOND_DIM * ATOM_DIM, ATOM_DIM), lambda b: (0, 0)),
        ],
        out_specs=pl.BlockSpec((C, N, ATOM_DIM), lambda b: (b, 0, 0)),
        out_shape=jax.ShapeDtypeStruct((B, N, ATOM_DIM), jnp.float32),
    )(atom_state, bond_state, src_idx, tgt_idx, w, r, r2)


# speedup vs baseline: 14.1893x; 1.4190x over previous
import jax
import jax.numpy as jnp
from jax.experimental import pallas as pl


B, N, E, ATOM_DIM, BOND_DIM = 64, 128, 256, 64, 16
C = 16
BF = jnp.bfloat16


def _bmm_kernel(atom_ref, bond_ref, src_ref, tgt_ref, w_ref, r_ref,
                out_ref):
    w = w_ref[...]

    iota_n = jax.lax.broadcasted_iota(jnp.int32, (E, N), 1)
    gathered = []
    for c in range(C):
        oh_src = (iota_n == src_ref[c, 0][:, None]).astype(BF)
        gathered.append(jax.lax.dot(oh_src, atom_ref[c],
                                    preferred_element_type=jnp.float32))
    src_atoms = jnp.concatenate(gathered, axis=0).astype(BF)

    bond = bond_ref[...].reshape(C * E, BOND_DIM)
    bond_exp = jax.lax.dot(bond, r_ref[...],
                           preferred_element_type=jnp.float32).astype(BF)
    g = jnp.tile(src_atoms, (1, BOND_DIM)) * bond_exp
    msg = jax.lax.dot(g, w, preferred_element_type=jnp.float32)
    msg = msg.astype(BF)

    iota_t = jax.lax.broadcasted_iota(jnp.int32, (N, E), 0)
    for c in range(C):
        oh_tgt = (iota_t == tgt_ref[c, 0][None, :]).astype(BF)
        out_ref[c] = jax.lax.dot(oh_tgt, msg[c * E:(c + 1) * E],
                                 preferred_element_type=jnp.float32)


@jax.jit
def kernel(atom_state, bond_state, connectivity, bond_transform):
    w = bond_transform.reshape(BOND_DIM, ATOM_DIM, ATOM_DIM)
    w = w.transpose(0, 2, 1).reshape(BOND_DIM * ATOM_DIM, ATOM_DIM)
    src_idx = connectivity[:, :, 0].reshape(B, 1, E)
    tgt_idx = connectivity[:, :, 1].reshape(B, 1, E)
    r = jnp.repeat(jnp.eye(BOND_DIM, dtype=BF), ATOM_DIM, axis=1)

    return pl.pallas_call(
        _bmm_kernel,
        grid=(B // C,),
        in_specs=[
            pl.BlockSpec((C, N, ATOM_DIM), lambda b: (b, 0, 0)),
            pl.BlockSpec((C, E, BOND_DIM), lambda b: (b, 0, 0)),
            pl.BlockSpec((C, 1, E), lambda b: (b, 0, 0)),
            pl.BlockSpec((C, 1, E), lambda b: (b, 0, 0)),
            pl.BlockSpec((BOND_DIM * ATOM_DIM, ATOM_DIM), lambda b: (0, 0)),
            pl.BlockSpec((BOND_DIM, BOND_DIM * ATOM_DIM), lambda b: (0, 0)),
        ],
        out_specs=pl.BlockSpec((C, N, ATOM_DIM), lambda b: (b, 0, 0)),
        out_shape=jax.ShapeDtypeStruct((B, N, ATOM_DIM), jnp.float32),
    )(atom_state.astype(BF), bond_state.astype(BF), src_idx, tgt_idx,
      w.astype(BF), r)
